# baseline (speedup 1.0000x reference)
import jax
import jax.numpy as jnp
from jax.experimental import pallas as pl
from jax.experimental.pallas import tpu as pltpu


def _se_kernel(x_ref, w1_ref, b1_ref, w2_ref, b2_ref, o_ref):
    x = x_ref[0]
    s = jnp.sum(x, axis=(1, 2))
    h = jnp.maximum(
        jnp.dot(w1_ref[...], s[:, None],
                preferred_element_type=jnp.float32)
        + b1_ref[...], 0.0)
    g = jax.nn.sigmoid(
        jnp.dot(w2_ref[...], h, preferred_element_type=jnp.float32)
        + b2_ref[...])
    o_ref[0] = (x * g[:, :, None]).astype(o_ref.dtype)


def kernel(x_nchw, fc1_w, fc1_b, fc2_w, fc2_b):
    B, C, H, W = x_nchw.shape
    Cr = fc1_w.shape[0]

    w1 = jnp.asarray(fc1_w, jnp.float32) * (1.0 / (H * W))
    b1 = jnp.asarray(fc1_b, jnp.float32).reshape(Cr, 1)
    w2 = jnp.asarray(fc2_w, jnp.float32)
    b2 = jnp.asarray(fc2_b, jnp.float32).reshape(C, 1)

    return pl.pallas_call(
        _se_kernel,
        out_shape=jax.ShapeDtypeStruct((B, C, H, W), x_nchw.dtype),
        grid=(B,),
        in_specs=[
            pl.BlockSpec((1, C, H, W), lambda b: (b, 0, 0, 0)),
            pl.BlockSpec((Cr, C), lambda b: (0, 0)),
            pl.BlockSpec((Cr, 1), lambda b: (0, 0)),
            pl.BlockSpec((C, Cr), lambda b: (0, 0)),
            pl.BlockSpec((C, 1), lambda b: (0, 0)),
        ],
        out_specs=pl.BlockSpec((1, C, H, W), lambda b: (b, 0, 0, 0)),
        compiler_params=pltpu.CompilerParams(
            dimension_semantics=("parallel",),
            vmem_limit_bytes=48 * 2**20),
        cost_estimate=pl.CostEstimate(
            flops=2 * B * C * H * W + 4 * B * C * Cr,
            transcendentals=B * C,
            bytes_accessed=2 * B * C * H * W * x_nchw.dtype.itemsize),
    )(x_nchw, w1, b1, w2, b2)


# speedup vs baseline: 2.0315x; 2.0315x over previous
import jax
import jax.numpy as jnp
from jax.experimental import pallas as pl
from jax.experimental.pallas import tpu as pltpu


def _sum_kernel(x_ref, o_ref):
    o_ref[...] = jnp.sum(x_ref[...], axis=(2, 3))[:, None, :]


def kernel(x_nchw, fc1_w, fc1_b, fc2_w, fc2_b):
    B, C, H, W = x_nchw.shape
    out = pl.pallas_call(
        _sum_kernel,
        out_shape=jax.ShapeDtypeStruct((B, 1, C), jnp.float32),
        grid=(B,),
        in_specs=[pl.BlockSpec((1, C, H, W), lambda b: (b, 0, 0, 0))],
        out_specs=pl.BlockSpec((1, 1, C), lambda b: (b, 0, 0)),
        compiler_params=pltpu.CompilerParams(
            dimension_semantics=("parallel",),
            vmem_limit_bytes=48 * 2**20),
    )(x_nchw)
    return out


# speedup vs baseline: 6.8205x; 3.3574x over previous
import jax
import jax.numpy as jnp
from jax.experimental import pallas as pl
from jax.experimental.pallas import tpu as pltpu


def _se_nhwc_kernel(x_ref, w1_ref, b1_ref, w2_ref, b2_ref, o_ref):
    x = x_ref[...]
    s = jnp.sum(x, axis=(1, 2))
    h = jnp.maximum(
        jnp.dot(s, w1_ref[...], preferred_element_type=jnp.float32)
        + b1_ref[...], 0.0)
    g = jax.nn.sigmoid(
        jnp.dot(h, w2_ref[...], preferred_element_type=jnp.float32)
        + b2_ref[...])
    o_ref[...] = (x * g[:, None, None, :]).astype(o_ref.dtype)


def kernel(x_nchw, fc1_w, fc1_b, fc2_w, fc2_b):
    B, C, H, W = x_nchw.shape
    Cr = fc1_w.shape[0]

    x = jnp.transpose(x_nchw, (0, 2, 3, 1))
    w1 = jnp.transpose(fc1_w) * (1.0 / (H * W))
    b1 = jnp.asarray(fc1_b, jnp.float32).reshape(1, Cr)
    w2 = jnp.transpose(fc2_w)
    b2 = jnp.asarray(fc2_b, jnp.float32).reshape(1, C)

    bt = 2
    out = pl.pallas_call(
        _se_nhwc_kernel,
        out_shape=jax.ShapeDtypeStruct((B, H, W, C), x.dtype),
        grid=(B // bt,),
        in_specs=[
            pl.BlockSpec((bt, H, W, C), lambda b: (b, 0, 0, 0)),
            pl.BlockSpec((C, Cr), lambda b: (0, 0)),
            pl.BlockSpec((1, Cr), lambda b: (0, 0)),
            pl.BlockSpec((Cr, C), lambda b: (0, 0)),
            pl.BlockSpec((1, C), lambda b: (0, 0)),
        ],
        out_specs=pl.BlockSpec((bt, H, W, C), lambda b: (b, 0, 0, 0)),
        compiler_params=pltpu.CompilerParams(
            dimension_semantics=("parallel",),
            vmem_limit_bytes=40 * 2**20),
        cost_estimate=pl.CostEstimate(
            flops=2 * B * C * H * W + 4 * B * C * Cr,
            transcendentals=B * C,
            bytes_accessed=2 * B * C * H * W * x.dtype.itemsize),
    )(x, w1, b1, w2, b2)
    return jnp.transpose(out, (0, 3, 1, 2))


# speedup vs baseline: 7.1607x; 1.0499x over previous
import jax
import jax.numpy as jnp
from jax.experimental import pallas as pl
from jax.experimental.pallas import tpu as pltpu


def _se_nhwc_kernel(x_ref, w1_ref, b1_ref, w2_ref, b2_ref, o_ref):
    x = x_ref[...]
    s = jnp.sum(x, axis=(1, 2))
    h = jnp.maximum(
        jnp.dot(s, w1_ref[...], preferred_element_type=jnp.float32)
        + b1_ref[...], 0.0)
    g = jax.nn.sigmoid(
        jnp.dot(h, w2_ref[...], preferred_element_type=jnp.float32)
        + b2_ref[...])
    o_ref[...] = (x * g[:, None, None, :]).astype(o_ref.dtype)


def kernel(x_nchw, fc1_w, fc1_b, fc2_w, fc2_b):
    B, C, H, W = x_nchw.shape
    Cr = fc1_w.shape[0]

    x = jnp.transpose(x_nchw, (0, 2, 3, 1))
    w1 = jnp.transpose(fc1_w) * (1.0 / (H * W))
    b1 = jnp.asarray(fc1_b, jnp.float32).reshape(1, Cr)
    w2 = jnp.transpose(fc2_w)
    b2 = jnp.asarray(fc2_b, jnp.float32).reshape(1, C)

    bt = 4
    out = pl.pallas_call(
        _se_nhwc_kernel,
        out_shape=jax.ShapeDtypeStruct((B, H, W, C), x.dtype),
        grid=(B // bt,),
        in_specs=[
            pl.BlockSpec((bt, H, W, C), lambda b: (b, 0, 0, 0)),
            pl.BlockSpec((C, Cr), lambda b: (0, 0)),
            pl.BlockSpec((1, Cr), lambda b: (0, 0)),
            pl.BlockSpec((Cr, C), lambda b: (0, 0)),
            pl.BlockSpec((1, C), lambda b: (0, 0)),
        ],
        out_specs=pl.BlockSpec((bt, H, W, C), lambda b: (b, 0, 0, 0)),
        compiler_params=pltpu.CompilerParams(
            dimension_semantics=("parallel",),
            vmem_limit_bytes=56 * 2**20),
        cost_estimate=pl.CostEstimate(
            flops=2 * B * C * H * W + 4 * B * C * Cr,
            transcendentals=B * C,
            bytes_accessed=2 * B * C * H * W * x.dtype.itemsize),
    )(x, w1, b1, w2, b2)
    return jnp.transpose(out, (0, 3, 1, 2))
